# Initial kernel scaffold; baseline (speedup 1.0000x reference)
import jax
import jax.numpy as jnp
from jax.experimental import pallas as pl


def kernel(x, pos_table, ln_gamma, ln_beta):
    raise NotImplementedError("write your pallas kernel here")


# speedup vs baseline: 2.1651x; 2.1651x over previous
import functools

import jax
import jax.numpy as jnp
from jax.experimental import pallas as pl
from jax.experimental.pallas import tpu as pltpu

EPS = 1e-5
BLOCK_S = 512


def _ln_kernel(x_ref, pos_ref, gamma_ref, beta_ref, out_ref):
    h = x_ref[0] + pos_ref[...]
    mean = jnp.mean(h, axis=-1, keepdims=True)
    c = h - mean
    var = jnp.mean(c * c, axis=-1, keepdims=True)
    out_ref[0] = c * jax.lax.rsqrt(var + EPS) * gamma_ref[...] + beta_ref[...]


@functools.partial(jax.jit, static_argnames=())
def kernel(x, pos_table, ln_gamma, ln_beta):
    B, S, D = x.shape
    bs = min(BLOCK_S, S)
    grid = (S // bs, B)
    return pl.pallas_call(
        _ln_kernel,
        grid=grid,
        in_specs=[
            pl.BlockSpec((1, bs, D), lambda s, b: (b, s, 0)),
            pl.BlockSpec((bs, D), lambda s, b: (s, 0)),
            pl.BlockSpec((D,), lambda s, b: (0,)),
            pl.BlockSpec((D,), lambda s, b: (0,)),
        ],
        out_specs=pl.BlockSpec((1, bs, D), lambda s, b: (b, s, 0)),
        out_shape=jax.ShapeDtypeStruct((B, S, D), x.dtype),
        compiler_params=pltpu.CompilerParams(
            dimension_semantics=("arbitrary", "arbitrary"),
        ),
    )(x, pos_table[:S], ln_gamma, ln_beta)


# speedup vs baseline: 2.4488x; 1.1310x over previous
import functools

import jax
import jax.numpy as jnp
from jax.experimental import pallas as pl
from jax.experimental.pallas import tpu as pltpu

EPS = 1e-5
BLOCK_S = 1024


def _ln_kernel(x_ref, pos_ref, gamma_ref, beta_ref, out_ref):
    h = x_ref[0] + pos_ref[...]
    mean = jnp.mean(h, axis=-1, keepdims=True)
    c = h - mean
    var = jnp.mean(c * c, axis=-1, keepdims=True)
    out_ref[0] = c * jax.lax.rsqrt(var + EPS) * gamma_ref[...] + beta_ref[...]


@functools.partial(jax.jit, static_argnames=())
def kernel(x, pos_table, ln_gamma, ln_beta):
    B, S, D = x.shape
    bs = min(BLOCK_S, S)
    grid = (S // bs, B)
    return pl.pallas_call(
        _ln_kernel,
        grid=grid,
        in_specs=[
            pl.BlockSpec((1, bs, D), lambda s, b: (b, s, 0)),
            pl.BlockSpec((bs, D), lambda s, b: (s, 0)),
            pl.BlockSpec((D,), lambda s, b: (0,)),
            pl.BlockSpec((D,), lambda s, b: (0,)),
        ],
        out_specs=pl.BlockSpec((1, bs, D), lambda s, b: (b, s, 0)),
        out_shape=jax.ShapeDtypeStruct((B, S, D), x.dtype),
        compiler_params=pltpu.CompilerParams(
            dimension_semantics=("arbitrary", "arbitrary"),
        ),
    )(x, pos_table[:S], ln_gamma, ln_beta)


# speedup vs baseline: 2.5818x; 1.0543x over previous
import functools

import jax
import jax.numpy as jnp
from jax.experimental import pallas as pl
from jax.experimental.pallas import tpu as pltpu

EPS = 1e-5
BLOCK_S = 2048


def _ln_kernel(x_ref, pos_ref, gamma_ref, beta_ref, out_ref):
    h = x_ref[0] + pos_ref[...]
    mean = jnp.mean(h, axis=-1, keepdims=True)
    c = h - mean
    var = jnp.mean(c * c, axis=-1, keepdims=True)
    out_ref[0] = c * jax.lax.rsqrt(var + EPS) * gamma_ref[...] + beta_ref[...]


@functools.partial(jax.jit, static_argnames=())
def kernel(x, pos_table, ln_gamma, ln_beta):
    B, S, D = x.shape
    bs = min(BLOCK_S, S)
    grid = (S // bs, B)
    return pl.pallas_call(
        _ln_kernel,
        grid=grid,
        in_specs=[
            pl.BlockSpec((1, bs, D), lambda s, b: (b, s, 0)),
            pl.BlockSpec((bs, D), lambda s, b: (s, 0)),
            pl.BlockSpec((D,), lambda s, b: (0,)),
            pl.BlockSpec((D,), lambda s, b: (0,)),
        ],
        out_specs=pl.BlockSpec((1, bs, D), lambda s, b: (b, s, 0)),
        out_shape=jax.ShapeDtypeStruct((B, S, D), x.dtype),
        compiler_params=pltpu.CompilerParams(
            dimension_semantics=("arbitrary", "arbitrary"),
        ),
    )(x, pos_table[:S], ln_gamma, ln_beta)


# speedup vs baseline: 2.6701x; 1.0342x over previous
import functools

import jax
import jax.numpy as jnp
from jax.experimental import pallas as pl
from jax.experimental.pallas import tpu as pltpu

EPS = 1e-5
BLOCK_S = 2048


def _ln_kernel(x_ref, pos_ref, gamma_ref, beta_ref, out_ref):
    h = x_ref[0] + pos_ref[...]
    d_inv = 1.0 / h.shape[-1]
    s1 = jnp.sum(h, axis=-1, keepdims=True)
    s2 = jnp.sum(h * h, axis=-1, keepdims=True)
    mean = s1 * d_inv
    var = s2 * d_inv - mean * mean
    inv = jax.lax.rsqrt(var + EPS)
    out_ref[0] = (h - mean) * inv * gamma_ref[...] + beta_ref[...]


@functools.partial(jax.jit, static_argnames=())
def kernel(x, pos_table, ln_gamma, ln_beta):
    B, S, D = x.shape
    bs = min(BLOCK_S, S)
    grid = (S // bs, B)
    return pl.pallas_call(
        _ln_kernel,
        grid=grid,
        in_specs=[
            pl.BlockSpec((1, bs, D), lambda s, b: (b, s, 0)),
            pl.BlockSpec((bs, D), lambda s, b: (s, 0)),
            pl.BlockSpec((D,), lambda s, b: (0,)),
            pl.BlockSpec((D,), lambda s, b: (0,)),
        ],
        out_specs=pl.BlockSpec((1, bs, D), lambda s, b: (b, s, 0)),
        out_shape=jax.ShapeDtypeStruct((B, S, D), x.dtype),
        compiler_params=pltpu.CompilerParams(
            dimension_semantics=("arbitrary", "arbitrary"),
        ),
    )(x, pos_table[:S], ln_gamma, ln_beta)


# speedup vs baseline: 2.7293x; 1.0222x over previous
import functools

import jax
import jax.numpy as jnp
from jax.experimental import pallas as pl
from jax.experimental.pallas import tpu as pltpu

EPS = 1e-5
BLOCK_S = 512


def _ln_kernel(x_ref, pos_ref, gamma_ref, beta_ref, out_ref):
    h = x_ref[...] + pos_ref[...][None]
    d_inv = 1.0 / h.shape[-1]
    s1 = jnp.sum(h, axis=-1, keepdims=True)
    s2 = jnp.sum(h * h, axis=-1, keepdims=True)
    mean = s1 * d_inv
    var = s2 * d_inv - mean * mean
    inv = jax.lax.rsqrt(var + EPS)
    out_ref[...] = (h - mean) * inv * gamma_ref[...] + beta_ref[...]


@functools.partial(jax.jit, static_argnames=())
def kernel(x, pos_table, ln_gamma, ln_beta):
    B, S, D = x.shape
    bs = min(BLOCK_S, S)
    grid = (S // bs,)
    return pl.pallas_call(
        _ln_kernel,
        grid=grid,
        in_specs=[
            pl.BlockSpec((B, bs, D), lambda s: (0, s, 0)),
            pl.BlockSpec((bs, D), lambda s: (s, 0)),
            pl.BlockSpec((D,), lambda s: (0,)),
            pl.BlockSpec((D,), lambda s: (0,)),
        ],
        out_specs=pl.BlockSpec((B, bs, D), lambda s: (0, s, 0)),
        out_shape=jax.ShapeDtypeStruct((B, S, D), x.dtype),
        compiler_params=pltpu.CompilerParams(
            dimension_semantics=("arbitrary",),
        ),
    )(x, pos_table[:S], ln_gamma, ln_beta)
